# baseline (speedup 1.0000x reference)
import functools

import jax
import jax.numpy as jnp
from jax import lax
from jax.experimental import pallas as pl
from jax.experimental.pallas import tpu as pltpu
from jax.experimental.pallas import tpu_sc as plsc


def _sc_gather_sum(B, HIST, D):
    info = plsc.get_sparse_core_info()
    nc, ns = info.num_cores, info.num_subcores
    nw = nc * ns
    assert B % nw == 0
    bpw = B // nw

    C0 = min(128, HIST)
    C1 = HIST - C0
    assert HIST % 8 == 0 and C0 % 8 == 0
    n_vec = D // 16

    mesh = plsc.VectorSubcoreMesh(core_axis_name="c", subcore_axis_name="s")

    @functools.partial(
        pl.kernel,
        mesh=mesh,
        compiler_params=pltpu.CompilerParams(use_tc_tiling_on_sc=False),
        out_type=jax.ShapeDtypeStruct((B, D), jnp.float32),
        scratch_types=[
            pltpu.VMEM((bpw * HIST,), jnp.int32),
            pltpu.VMEM((HIST, D), jnp.float32),
            pltpu.VMEM((bpw, D), jnp.float32),
            pltpu.SemaphoreType.DMA,
            pltpu.SemaphoreType.DMA,
        ],
    )
    def sc_sum(idx_hbm, table_hbm, out_hbm, idx_v, rows_v, stage_v, sem0, sem1):
        wid = lax.axis_index("s") * nc + lax.axis_index("c")
        base = wid * bpw
        pltpu.sync_copy(idx_hbm.at[pl.ds(base * HIST, bpw * HIST)], idx_v)

        def row_body(r, carry):
            off = pl.multiple_of(r * HIST, 8)
            cp0 = pltpu.async_copy(
                table_hbm.at[idx_v.at[pl.ds(off, C0)]],
                rows_v.at[pl.ds(0, C0)], sem0)
            if C1:
                cp1 = pltpu.async_copy(
                    table_hbm.at[idx_v.at[pl.ds(off + C0, C1)]],
                    rows_v.at[pl.ds(C0, C1)], sem1)
            cp0.wait()
            if C1:
                cp1.wait()

            def acc_body(j, accs):
                return tuple(accs[k] + rows_v[j, pl.ds(16 * k, 16)]
                             for k in range(n_vec))

            zero = jnp.zeros((16,), jnp.float32)
            accs = lax.fori_loop(0, HIST, acc_body, (zero,) * n_vec)
            for k in range(n_vec):
                stage_v[r, pl.ds(16 * k, 16)] = accs[k]
            return carry

        lax.fori_loop(0, bpw, row_body, 0)
        pltpu.sync_copy(stage_v, out_hbm.at[pl.ds(base, bpw)])

    return sc_sum


def _tc_linear(sums, W, b2, scale):
    B, D = sums.shape
    OUT = W.shape[1]
    blk = 512 if B % 512 == 0 else B

    def body(s_ref, w_ref, b_ref, o_ref):
        o_ref[...] = jnp.dot(s_ref[...] * scale, w_ref[...],
                             preferred_element_type=jnp.float32) + b_ref[...]

    return pl.pallas_call(
        body,
        grid=(B // blk,),
        in_specs=[
            pl.BlockSpec((blk, D), lambda i: (i, 0)),
            pl.BlockSpec((D, OUT), lambda i: (0, 0)),
            pl.BlockSpec((1, OUT), lambda i: (0, 0)),
        ],
        out_specs=pl.BlockSpec((blk, OUT), lambda i: (i, 0)),
        out_shape=jax.ShapeDtypeStruct((B, OUT), jnp.float32),
    )(sums, W, b2)


def kernel(word_indices, embedding, W, b):
    B, HIST = word_indices.shape
    D = embedding.shape[1]
    idx_flat = word_indices.reshape(-1).astype(jnp.int32)
    sums = _sc_gather_sum(B, HIST, D)(idx_flat, embedding)
    return _tc_linear(sums, W, b.reshape(1, -1), 1.0 / HIST)


# speedup vs baseline: 1.1853x; 1.1853x over previous
import functools

import jax
import jax.numpy as jnp
from jax import lax
from jax.experimental import pallas as pl
from jax.experimental.pallas import tpu as pltpu
from jax.experimental.pallas import tpu_sc as plsc


def _sc_gather_sum(B, HIST, D):
    info = plsc.get_sparse_core_info()
    nc, ns = info.num_cores, info.num_subcores
    nw = nc * ns
    assert B % nw == 0
    bpw = B // nw

    n_vec = D // 16

    G = 2
    NBUF = 2
    GH = G * HIST
    NG = bpw // G
    U = 8
    assert bpw % (G * NBUF) == 0 and HIST % U == 0 and GH % 8 == 0
    chunks = [(o, min(128, GH - o)) for o in range(0, GH, 128)]
    assert all(o % 8 == 0 for o, _ in chunks)

    mesh = plsc.VectorSubcoreMesh(core_axis_name="c", subcore_axis_name="s")

    @functools.partial(
        pl.kernel,
        mesh=mesh,
        compiler_params=pltpu.CompilerParams(use_tc_tiling_on_sc=False),
        out_type=jax.ShapeDtypeStruct((B, D), jnp.float32),
        scratch_types=[
            pltpu.VMEM((bpw * HIST,), jnp.int32),
            pltpu.VMEM((NBUF, GH, D), jnp.float32),
            pltpu.VMEM((bpw, D), jnp.float32),
        ] + [pltpu.SemaphoreType.DMA] * NBUF,
    )
    def sc_sum(idx_hbm, table_hbm, out_hbm, idx_v, rows_v, stage_v, *sems):
        wid = lax.axis_index("s") * nc + lax.axis_index("c")
        base = wid * bpw
        pltpu.sync_copy(idx_hbm.at[pl.ds(base * HIST, bpw * HIST)], idx_v)

        def _copies(g, buf):
            off = pl.multiple_of(g * GH, 8)
            return [pltpu.make_async_copy(
                        table_hbm.at[idx_v.at[pl.ds(off + co, cl)]],
                        rows_v.at[buf, pl.ds(co, cl)],
                        sems[buf])
                    for co, cl in chunks]

        def issue(g, buf):
            for c in _copies(g, buf):
                c.start()

        def drain(g, buf):
            for c in _copies(g, buf):
                c.wait()

        zero = jnp.zeros((16,), jnp.float32)

        def accum(g, buf):
            for rr in range(G):
                def body(jj, accs, _rr=rr):
                    j0 = _rr * HIST + jj * U
                    for u in range(U):
                        accs = tuple(
                            accs[k] + rows_v[buf, j0 + u, pl.ds(16 * k, 16)]
                            for k in range(n_vec))
                    return accs

                accs = lax.fori_loop(0, HIST // U, body, (zero,) * n_vec)
                r_out = g * G + rr
                for k in range(n_vec):
                    stage_v[r_out, pl.ds(16 * k, 16)] = accs[k]

        issue(0, 0)

        def outer(i, carry):
            g0 = i * NBUF
            for b in range(NBUF):
                cur = g0 + b
                nxt = cur + 1

                @pl.when(nxt < NG)
                def _(nxt=nxt, b=b):
                    issue(nxt, (b + 1) % NBUF)

                drain(cur, b)
                accum(cur, b)
            return carry

        lax.fori_loop(0, NG // NBUF, outer, 0)
        pltpu.sync_copy(stage_v, out_hbm.at[pl.ds(base, bpw)])

    return sc_sum


def _tc_linear(sums, W, b2, scale):
    B, D = sums.shape
    OUT = W.shape[1]
    blk = 512 if B % 512 == 0 else B

    def body(s_ref, w_ref, b_ref, o_ref):
        o_ref[...] = jnp.dot(s_ref[...] * scale, w_ref[...],
                             preferred_element_type=jnp.float32) + b_ref[...]

    return pl.pallas_call(
        body,
        grid=(B // blk,),
        in_specs=[
            pl.BlockSpec((blk, D), lambda i: (i, 0)),
            pl.BlockSpec((D, OUT), lambda i: (0, 0)),
            pl.BlockSpec((1, OUT), lambda i: (0, 0)),
        ],
        out_specs=pl.BlockSpec((blk, OUT), lambda i: (i, 0)),
        out_shape=jax.ShapeDtypeStruct((B, OUT), jnp.float32),
    )(sums, W, b2)


def kernel(word_indices, embedding, W, b):
    B, HIST = word_indices.shape
    D = embedding.shape[1]
    idx_flat = word_indices.reshape(-1).astype(jnp.int32)
    sums = _sc_gather_sum(B, HIST, D)(idx_flat, embedding)
    return _tc_linear(sums, W, b.reshape(1, -1), 1.0 / HIST)
